# baseline (speedup 1.0000x reference)
import functools

import jax
import jax.numpy as jnp
from jax import lax
from jax.experimental import pallas as pl
from jax.experimental.pallas import tpu as pltpu
from jax.experimental.pallas import tpu_sc as plsc

_B = 16
_L = 512
_VOCAB = 30000
_TOTAL = 4096
_CLS = _VOCAB + 1
_SEP = _VOCAB + 2
_LANES = 16
_COLS_PER_TILE = _L // 2


def _make_kernel():
    mesh = plsc.VectorSubcoreMesh(
        core_axis_name="c", subcore_axis_name="s", num_cores=1
    )

    @functools.partial(
        pl.kernel,
        mesh=mesh,
        out_type=(
            jax.ShapeDtypeStruct((_B, _L), jnp.int32),
            jax.ShapeDtypeStruct((_B, _L), jnp.int32),
        ),
        compiler_params=pltpu.CompilerParams(needs_layout_passes=False),
        scratch_types=[
            pltpu.VMEM((_TOTAL,), jnp.int32),
            pltpu.VMEM((_TOTAL,), jnp.int32),
            pltpu.VMEM((_L,), jnp.int32),
            pltpu.VMEM((_L,), jnp.int32),
            pltpu.SemaphoreType.DMA,
            pltpu.SemaphoreType.DMA,
            pltpu.SemaphoreType.DMA,
            pltpu.SemaphoreType.DMA,
        ],
    )
    def tok_kernel(
        vals_hbm, seg_hbm, out_hbm, segout_hbm, vals_v, seg_v, out_v, zero_v,
        sem_s, sem_v, sem_z, sem_o
    ):
        row = lax.axis_index("s")

        cp_seg = pltpu.async_copy(seg_hbm, seg_v, sem_s)
        cp_vals = pltpu.async_copy(vals_hbm, vals_v, sem_v)

        def zero_body(v, carry):
            zero_v[pl.ds(v * _LANES, _LANES)] = jnp.zeros((_LANES,), jnp.int32)
            return carry

        lax.fori_loop(0, _L // _LANES, zero_body, 0)
        cp_zero = pltpu.async_copy(zero_v, segout_hbm.at[row], sem_z)

        cp_seg.wait()

        lane = lax.iota(jnp.int32, _LANES)

        def count_lt(r):
            base = jnp.int32(0)
            for step in (256, 16, 1):
                idx = base + lane * step + (step - 1)
                sv = plsc.load_gather(seg_v, [jnp.clip(idx, 0, _TOTAL - 1)])
                ok = jnp.where((sv < r) & (idx < _TOTAL), 1, 0).astype(jnp.int32)
                base = base + jnp.sum(ok) * step
            return base

        offs = count_lt(row)
        length = count_lt(row + 1) - offs
        cp_vals.wait()

        def out_body(v, carry):
            col = v * _LANES + lane
            pos = col - 1
            gidx = jnp.clip(offs + pos, 0, _TOTAL - 1)
            val = plsc.load_gather(vals_v, [gidx])
            res = jnp.where(
                col == 0,
                _CLS,
                jnp.where(pos < length, val, jnp.where(pos == length, _SEP, 0)),
            ).astype(jnp.int32)
            out_v[pl.ds(v * _LANES, _LANES)] = res
            return carry

        lax.fori_loop(0, _L // _LANES, out_body, 0)

        cp_out = pltpu.async_copy(out_v, out_hbm.at[row], sem_o)
        cp_out.wait()
        cp_zero.wait()

    return tok_kernel


_tok = _make_kernel()


@jax.jit
def kernel(pieces_vals, segment_ids):
    tokens, segments = _tok(pieces_vals, segment_ids)
    return (tokens, segments)
